# baseline (speedup 1.0000x reference)
import jax
import jax.numpy as jnp
from jax.experimental import pallas as pl
from jax.experimental.pallas import tpu as pltpu

_HBM = pltpu.MemorySpace.HBM
_K = 16


def _copy_body(u_in, i_in, u_out, i_out, sem_u, sem_i):
    cu = u_in.shape[0] // _K
    ci = i_in.shape[0] // _K
    for k in range(_K):
        pltpu.make_async_copy(
            u_in.at[pl.ds(k * cu, cu)], u_out.at[pl.ds(k * cu, cu)], sem_u.at[k]
        ).start()
        pltpu.make_async_copy(
            i_in.at[pl.ds(k * ci, ci)], i_out.at[pl.ds(k * ci, ci)], sem_i.at[k]
        ).start()
    for k in range(_K):
        pltpu.make_async_copy(
            u_in.at[pl.ds(k * cu, cu)], u_out.at[pl.ds(k * cu, cu)], sem_u.at[k]
        ).wait()
        pltpu.make_async_copy(
            i_in.at[pl.ds(k * ci, ci)], i_out.at[pl.ds(k * ci, ci)], sem_i.at[k]
        ).wait()


def kernel(edge_index, user_weight, item_weight):
    u_out, i_out = pl.pallas_call(
        _copy_body,
        in_specs=[
            pl.BlockSpec(memory_space=_HBM),
            pl.BlockSpec(memory_space=_HBM),
        ],
        out_specs=[
            pl.BlockSpec(memory_space=_HBM),
            pl.BlockSpec(memory_space=_HBM),
        ],
        out_shape=[
            jax.ShapeDtypeStruct(user_weight.shape, user_weight.dtype),
            jax.ShapeDtypeStruct(item_weight.shape, item_weight.dtype),
        ],
        scratch_shapes=[
            pltpu.SemaphoreType.DMA((_K,)),
            pltpu.SemaphoreType.DMA((_K,)),
        ],
    )(user_weight, item_weight)
    return (u_out, i_out)


# speedup vs baseline: 16.1525x; 16.1525x over previous
import jax
import jax.numpy as jnp
from jax.experimental import pallas as pl
from jax.experimental.pallas import tpu as pltpu

_HBM = pltpu.MemorySpace.HBM
_CH = 5000
_D = 12
_H = 6


def _copy_body(u_in, i_in, u_out, i_out, bufs, in_sems, out_sems):
    chunks = [(u_in, u_out, k) for k in range(u_in.shape[0] // _CH)]
    chunks += [(i_in, i_out, k) for k in range(i_in.shape[0] // _CH)]
    n = len(chunks)

    def in_copy(c):
        src, _, k = chunks[c]
        b = c % _D
        return pltpu.make_async_copy(
            src.at[pl.ds(k * _CH, _CH)], bufs.at[b], in_sems.at[b]
        )

    def out_copy(c):
        _, dst, k = chunks[c]
        b = c % _D
        return pltpu.make_async_copy(
            bufs.at[b], dst.at[pl.ds(k * _CH, _CH)], out_sems.at[b]
        )

    for c in range(n):
        if c >= _D:
            out_copy(c - _D).wait()
        in_copy(c).start()
        if c >= _H:
            in_copy(c - _H).wait()
            out_copy(c - _H).start()
    for c in range(n - _H, n):
        in_copy(c).wait()
        out_copy(c).start()
    for c in range(n - _D, n):
        out_copy(c).wait()


def kernel(edge_index, user_weight, item_weight):
    d = user_weight.shape[1]
    u_out, i_out = pl.pallas_call(
        _copy_body,
        in_specs=[
            pl.BlockSpec(memory_space=_HBM),
            pl.BlockSpec(memory_space=_HBM),
        ],
        out_specs=[
            pl.BlockSpec(memory_space=_HBM),
            pl.BlockSpec(memory_space=_HBM),
        ],
        out_shape=[
            jax.ShapeDtypeStruct(user_weight.shape, user_weight.dtype),
            jax.ShapeDtypeStruct(item_weight.shape, item_weight.dtype),
        ],
        scratch_shapes=[
            pltpu.VMEM((_D, _CH, d), jnp.float32),
            pltpu.SemaphoreType.DMA((_D,)),
            pltpu.SemaphoreType.DMA((_D,)),
        ],
    )(user_weight, item_weight)
    return (u_out, i_out)
